# baseline (speedup 1.0000x reference)
import functools

import jax
import jax.numpy as jnp
from jax import lax
from jax.experimental import pallas as pl
from jax.experimental.pallas import tpu as pltpu
from jax.experimental.pallas import tpu_sc as plsc

_BETA = 15.0
_BATCH = 16384
_D = 16
_N_ITEMS = 100000
_NW = 32
_BPW = _BATCH // _NW
_CHUNK = 16
_OCHUNK = 128
_NOC = _BPW // _OCHUNK
_IPO = _OCHUNK // _CHUNK


def _sc_kernel(user_hbm, item_hbm, ideal_hbm, mid_hbm, spread_hbm, out_hbm,
               uidx_v, ilin_v, ilane_v, blk0_v, blk1_v, mline_v, sline_v,
               out_v, sem0, sem1, semi):
    wid = lax.axis_index("s") * 2 + lax.axis_index("c")
    base = wid * _BPW

    pltpu.sync_copy(user_hbm.at[pl.ds(base, _BPW)], uidx_v)
    pltpu.sync_copy(item_hbm.at[pl.ds(base, _BPW)], ilin_v)

    def split(k, _):
        sl = pl.ds(k * 16, 16)
        t = ilin_v[sl]
        ilane_v[sl] = (t & 7) * _D
        ilin_v[sl] = t >> 3
        return _

    lax.fori_loop(0, _BPW // 16, split, None)

    lane = lax.iota(jnp.int32, 16)

    def issue(c, blk, sem):
        ublk = (uidx_v[pl.ds(c * _CHUNK, _CHUNK)] >> 7) * 128
        for j in range(_CHUNK):
            off = pl.multiple_of(ublk[j], 128)
            pltpu.async_copy(
                ideal_hbm.at[:, pl.ds(off, 128)], blk.at[j], sem)

    def drain(blk, sem):
        for j in range(_CHUNK):
            pltpu.make_async_copy(
                ideal_hbm.at[:, pl.ds(0, 128)], blk.at[j], sem).wait()

    issue(0, blk0_v, sem0)
    issue(1, blk1_v, sem1)

    for oc in range(_NOC):
        cp_m = pltpu.async_copy(
            mid_hbm.at[ilin_v.at[pl.ds(oc * _OCHUNK, _OCHUNK)]],
            mline_v, semi)
        cp_s = pltpu.async_copy(
            spread_hbm.at[ilin_v.at[pl.ds(oc * _OCHUNK, _OCHUNK)]],
            sline_v, semi)
        cp_m.wait()
        cp_s.wait()

        for ic in range(_IPO):
            c = oc * _IPO + ic
            blk, sem = (blk0_v, sem0) if c % 2 == 0 else (blk1_v, sem1)
            drain(blk, sem)

            csl = pl.ds(c * _CHUNK, _CHUNK)
            ulane = uidx_v[csl] & 127
            ilane = ilane_v[csl]
            erow = ic * _CHUNK + lane
            acc = jnp.zeros((16,), jnp.float32)
            for d in range(_D):
                dv = jnp.full((16,), d, jnp.int32)
                xv = plsc.load_gather(blk, [lane, dv, ulane])
                mv = plsc.load_gather(mline_v, [erow, ilane + d])
                sv = plsc.load_gather(sline_v, [erow, ilane + d])
                acc = acc + (xv - mv) * sv
            out_v[csl] = acc * _BETA

            if c + 2 < _NCHUNK_TOTAL:
                issue(c + 2, blk, sem)

    pltpu.sync_copy(out_v, out_hbm.at[pl.ds(base, _BPW)])


_NCHUNK_TOTAL = _BPW // _CHUNK


@jax.jit
def kernel(user_idx, item_idx, ideal_points, vote_midpoints, vote_spreads):
    mesh = plsc.VectorSubcoreMesh(core_axis_name="c", subcore_axis_name="s")
    run = functools.partial(
        pl.kernel,
        mesh=mesh,
        out_type=jax.ShapeDtypeStruct((_BATCH,), jnp.float32),
        scratch_types=[
            pltpu.VMEM((_BPW,), jnp.int32),
            pltpu.VMEM((_BPW,), jnp.int32),
            pltpu.VMEM((_BPW,), jnp.int32),
            pltpu.VMEM((_CHUNK, _D, 128), jnp.float32),
            pltpu.VMEM((_CHUNK, _D, 128), jnp.float32),
            pltpu.VMEM((_OCHUNK, 128), jnp.float32),
            pltpu.VMEM((_OCHUNK, 128), jnp.float32),
            pltpu.VMEM((_BPW,), jnp.float32),
            pltpu.SemaphoreType.DMA,
            pltpu.SemaphoreType.DMA,
            pltpu.SemaphoreType.DMA,
        ],
        compiler_params=pltpu.CompilerParams(needs_layout_passes=False),
    )(_sc_kernel)
    return run(user_idx.astype(jnp.int32), item_idx.astype(jnp.int32),
               ideal_points.T,
               vote_midpoints.reshape(_N_ITEMS // 8, 128),
               vote_spreads.reshape(_N_ITEMS // 8, 128))


# speedup vs baseline: 1.2140x; 1.2140x over previous
import functools

import jax
import jax.numpy as jnp
from jax import lax
from jax.experimental import pallas as pl
from jax.experimental.pallas import tpu as pltpu
from jax.experimental.pallas import tpu_sc as plsc

_BETA = 15.0
_BATCH = 16384
_D = 16
_N_USERS = 1000000
_N_ITEMS = 100000
_NW = 32
_BPW = _BATCH // _NW
_CHUNK = 16
_NCHUNK = _BPW // _CHUNK


def _gather_x_kernel(user_hbm, table_hbm, xflat_hbm,
                     uidx_v, blk0_v, blk1_v, x_v, sem0, sem1):
    wid = lax.axis_index("s") * 2 + lax.axis_index("c")
    base = wid * _BPW

    pltpu.sync_copy(user_hbm.at[pl.ds(base, _BPW)], uidx_v)

    lane = lax.iota(jnp.int32, 16)

    def issue(c, blk, sem):
        ublk = (uidx_v[pl.ds(c * _CHUNK, _CHUNK)] >> 7) * 128
        for j in range(_CHUNK):
            off = pl.multiple_of(ublk[j], 128)
            for h in range(2):
                pltpu.async_copy(
                    table_hbm.at[pl.ds(h * 8, 8), pl.ds(off, 128)],
                    blk.at[j, h], sem)

    def drain(blk, sem):
        for j in range(_CHUNK):
            for h in range(2):
                pltpu.make_async_copy(
                    table_hbm.at[pl.ds(0, 8), pl.ds(0, 128)],
                    blk.at[j, h], sem).wait()

    def extract(c, blk):
        ulane = uidx_v[pl.ds(c * _CHUNK, _CHUNK)] & 127
        ebase = (c * _CHUNK + lane) * _D
        for d in range(_D):
            hv = jnp.full((16,), d // 8, jnp.int32)
            dv = jnp.full((16,), d % 8, jnp.int32)
            xv = plsc.load_gather(blk, [lane, hv, dv, ulane])
            plsc.store_scatter(x_v, [ebase + d], xv)

    issue(0, blk0_v, sem0)

    for c in range(_NCHUNK):
        blk, sem = (blk0_v, sem0) if c % 2 == 0 else (blk1_v, sem1)
        nblk, nsem = (blk1_v, sem1) if c % 2 == 0 else (blk0_v, sem0)
        if c + 1 < _NCHUNK:
            issue(c + 1, nblk, nsem)
        drain(blk, sem)
        extract(c, blk)

    pltpu.sync_copy(x_v, xflat_hbm.at[pl.ds(base * _D, _BPW * _D)])


def _dot_kernel(item_hbm, xflat_hbm, mid_hbm, spread_hbm, out_hbm,
                iidx_v, x_v, m_v, s_v, out_v, sem):
    wid = lax.axis_index("s") * 2 + lax.axis_index("c")
    base = wid * _BPW

    pltpu.sync_copy(item_hbm.at[pl.ds(base, _BPW)], iidx_v)
    cp_x = pltpu.async_copy(
        xflat_hbm.at[pl.ds(base * _D, _BPW * _D)], x_v, sem)
    cp_m = pltpu.async_copy(mid_hbm.at[iidx_v], m_v, sem)
    cp_s = pltpu.async_copy(spread_hbm.at[iidx_v], s_v, sem)
    cp_x.wait()
    cp_m.wait()
    cp_s.wait()

    lane = lax.iota(jnp.int32, 16)

    def body(g, _):
        rows = g * 16 + lane
        flat = rows * _D
        acc = jnp.zeros((16,), jnp.float32)
        for d in range(_D):
            cols = jnp.full((16,), d, jnp.int32)
            xv = plsc.load_gather(x_v, [flat + d])
            mv = plsc.load_gather(m_v, [rows, cols])
            sv = plsc.load_gather(s_v, [rows, cols])
            acc = acc + (xv - mv) * sv
        out_v[pl.ds(g * 16, 16)] = acc * _BETA
        return _

    lax.fori_loop(0, _BPW // 16, body, None)

    pltpu.sync_copy(out_v, out_hbm.at[pl.ds(base, _BPW)])


@jax.jit
def kernel(user_idx, item_idx, ideal_points, vote_midpoints, vote_spreads):
    mesh = plsc.VectorSubcoreMesh(core_axis_name="c", subcore_axis_name="s")

    gather_x = functools.partial(
        pl.kernel,
        mesh=mesh,
        out_type=jax.ShapeDtypeStruct((_BATCH * _D,), jnp.float32),
        scratch_types=[
            pltpu.VMEM((_BPW,), jnp.int32),
            pltpu.VMEM((_CHUNK, 2, 8, 128), jnp.float32),
            pltpu.VMEM((_CHUNK, 2, 8, 128), jnp.float32),
            pltpu.VMEM((_BPW * _D,), jnp.float32),
            pltpu.SemaphoreType.DMA,
            pltpu.SemaphoreType.DMA,
        ],
        compiler_params=pltpu.CompilerParams(needs_layout_passes=False),
    )(_gather_x_kernel)
    xflat = gather_x(user_idx.astype(jnp.int32), ideal_points.T)

    dot = functools.partial(
        pl.kernel,
        mesh=mesh,
        out_type=jax.ShapeDtypeStruct((_BATCH,), jnp.float32),
        scratch_types=[
            pltpu.VMEM((_BPW,), jnp.int32),
            pltpu.VMEM((_BPW * _D,), jnp.float32),
            pltpu.VMEM((_BPW, _D), jnp.float32),
            pltpu.VMEM((_BPW, _D), jnp.float32),
            pltpu.VMEM((_BPW,), jnp.float32),
            pltpu.SemaphoreType.DMA,
        ],
        compiler_params=pltpu.CompilerParams(
            needs_layout_passes=False, use_tc_tiling_on_sc=False),
    )(_dot_kernel)
    return dot(item_idx.astype(jnp.int32), xflat,
               vote_midpoints, vote_spreads)


# speedup vs baseline: 1.2257x; 1.0096x over previous
import functools

import jax
import jax.numpy as jnp
from jax import lax
from jax.experimental import pallas as pl
from jax.experimental.pallas import tpu as pltpu
from jax.experimental.pallas import tpu_sc as plsc

_BETA = 15.0
_BATCH = 16384
_D = 16
_N_USERS = 1000000
_N_ITEMS = 100000
_NW = 32
_BPW = _BATCH // _NW
_CHUNK = 16
_NCHUNK = _BPW // _CHUNK


def _gather_x_kernel(user_hbm, table_hbm, xflat_hbm,
                     uidx_v, blk0_v, blk1_v, blk2_v, x_v, sem0, sem1, sem2):
    wid = lax.axis_index("s") * 2 + lax.axis_index("c")
    base = wid * _BPW

    pltpu.sync_copy(user_hbm.at[pl.ds(base, _BPW)], uidx_v)

    lane = lax.iota(jnp.int32, 16)

    def issue(c, blk, sem):
        ublk = (uidx_v[pl.ds(c * _CHUNK, _CHUNK)] >> 7) * 128
        for j in range(_CHUNK):
            off = pl.multiple_of(ublk[j], 128)
            pltpu.async_copy(
                table_hbm.at[:, pl.ds(off, 128)], blk.at[j], sem)

    def drain(blk, sem):
        for j in range(_CHUNK):
            pltpu.make_async_copy(
                table_hbm.at[:, pl.ds(0, 128)], blk.at[j], sem).wait()

    def extract(c, blk):
        ulane = uidx_v[pl.ds(c * _CHUNK, _CHUNK)] & 127
        ebase = (c * _CHUNK + lane) * _D
        for d in range(_D):
            dv = jnp.full((16,), d, jnp.int32)
            xv = plsc.load_gather(blk, [lane, dv, ulane])
            plsc.store_scatter(x_v, [ebase + d], xv)

    bufs = [(blk0_v, sem0), (blk1_v, sem1), (blk2_v, sem2)]
    issue(0, *bufs[0])
    issue(1, *bufs[1])
    issue(2, *bufs[2])

    for c in range(_NCHUNK):
        blk, sem = bufs[c % 3]
        drain(blk, sem)
        extract(c, blk)
        if c + 3 < _NCHUNK:
            issue(c + 3, blk, sem)

    pltpu.sync_copy(x_v, xflat_hbm.at[pl.ds(base * _D, _BPW * _D)])


def _dot_kernel(item_hbm, xflat_hbm, mid_hbm, spread_hbm, out_hbm,
                iidx_v, x_v, m_v, s_v, out_v, sem):
    wid = lax.axis_index("s") * 2 + lax.axis_index("c")
    base = wid * _BPW

    pltpu.sync_copy(item_hbm.at[pl.ds(base, _BPW)], iidx_v)
    cp_x = pltpu.async_copy(
        xflat_hbm.at[pl.ds(base * _D, _BPW * _D)], x_v, sem)
    cp_m = pltpu.async_copy(mid_hbm.at[iidx_v], m_v, sem)
    cp_s = pltpu.async_copy(spread_hbm.at[iidx_v], s_v, sem)
    cp_x.wait()
    cp_m.wait()
    cp_s.wait()

    lane = lax.iota(jnp.int32, 16)

    def body(g, _):
        rows = g * 16 + lane
        flat = rows * _D
        acc = jnp.zeros((16,), jnp.float32)
        for d in range(_D):
            cols = jnp.full((16,), d, jnp.int32)
            xv = plsc.load_gather(x_v, [flat + d])
            mv = plsc.load_gather(m_v, [rows, cols])
            sv = plsc.load_gather(s_v, [rows, cols])
            acc = acc + (xv - mv) * sv
        out_v[pl.ds(g * 16, 16)] = acc * _BETA
        return _

    lax.fori_loop(0, _BPW // 16, body, None)

    pltpu.sync_copy(out_v, out_hbm.at[pl.ds(base, _BPW)])


@jax.jit
def kernel(user_idx, item_idx, ideal_points, vote_midpoints, vote_spreads):
    mesh = plsc.VectorSubcoreMesh(core_axis_name="c", subcore_axis_name="s")

    gather_x = functools.partial(
        pl.kernel,
        mesh=mesh,
        out_type=jax.ShapeDtypeStruct((_BATCH * _D,), jnp.float32),
        scratch_types=[
            pltpu.VMEM((_BPW,), jnp.int32),
            pltpu.VMEM((_CHUNK, _D, 128), jnp.float32),
            pltpu.VMEM((_CHUNK, _D, 128), jnp.float32),
            pltpu.VMEM((_CHUNK, _D, 128), jnp.float32),
            pltpu.VMEM((_BPW * _D,), jnp.float32),
            pltpu.SemaphoreType.DMA,
            pltpu.SemaphoreType.DMA,
            pltpu.SemaphoreType.DMA,
        ],
        compiler_params=pltpu.CompilerParams(needs_layout_passes=False),
    )(_gather_x_kernel)
    xflat = gather_x(user_idx.astype(jnp.int32), ideal_points.T)

    dot = functools.partial(
        pl.kernel,
        mesh=mesh,
        out_type=jax.ShapeDtypeStruct((_BATCH,), jnp.float32),
        scratch_types=[
            pltpu.VMEM((_BPW,), jnp.int32),
            pltpu.VMEM((_BPW * _D,), jnp.float32),
            pltpu.VMEM((_BPW, _D), jnp.float32),
            pltpu.VMEM((_BPW, _D), jnp.float32),
            pltpu.VMEM((_BPW,), jnp.float32),
            pltpu.SemaphoreType.DMA,
        ],
        compiler_params=pltpu.CompilerParams(
            needs_layout_passes=False, use_tc_tiling_on_sc=False),
    )(_dot_kernel)
    return dot(item_idx.astype(jnp.int32), xflat,
               vote_midpoints, vote_spreads)
